# baseline (speedup 1.0000x reference)
import functools

import jax
import jax.numpy as jnp
from jax.experimental import pallas as pl
from jax.experimental.pallas import tpu as pltpu
from jax.experimental.pallas import tpu_sc as plsc

_VOCAB_PAD = 128
_EMBED = 512
_WINDOW = 128


def _project_body(t_ref, w_ref, b_ref, o_ref):
    o_ref[...] = (
        jnp.dot(t_ref[...], w_ref[...], preferred_element_type=jnp.float32)
        + b_ref[...]
    )


def _project(table_pad, W, b2d):
    return pl.pallas_call(
        _project_body,
        out_shape=jax.ShapeDtypeStruct((_VOCAB_PAD, _EMBED), jnp.float32),
    )(table_pad, W, b2d)


_NW = 32


def _gather(P, idx):
    n = idx.shape[0]
    per_w = n // _NW
    mesh = plsc.VectorSubcoreMesh(core_axis_name="core", subcore_axis_name="subcore")

    @functools.partial(
        pl.kernel,
        out_type=jax.ShapeDtypeStruct((n, _EMBED), jnp.float32),
        mesh=mesh,
        scratch_types=[
            pltpu.VMEM((_WINDOW,), jnp.int32),
            pltpu.VMEM((_WINDOW, _EMBED), jnp.float32),
        ],
    )
    def k(p_hbm, i_hbm, o_hbm, idx_v, rows_v):
        wid = jax.lax.axis_index("subcore") * 2 + jax.lax.axis_index("core")
        base = wid * per_w

        @pl.loop(0, per_w, step=_WINDOW)
        def _(g):
            off = base + g
            pltpu.sync_copy(i_hbm.at[pl.ds(off, _WINDOW)], idx_v)
            pltpu.sync_copy(p_hbm.at[idx_v], rows_v)
            pltpu.sync_copy(rows_v, o_hbm.at[pl.ds(off, _WINDOW)])

    return k(P, idx)


def kernel(indices, table, W, b):
    B, S = indices.shape
    table_pad = jnp.pad(table, ((0, _VOCAB_PAD - table.shape[0]), (0, 0)))
    P = _project(table_pad, W, b.reshape(1, _EMBED))
    idx = indices.reshape(B * S).astype(jnp.int32)
    out = _gather(P, idx)
    return out.reshape(B, S, _EMBED)
